# Initial kernel scaffold; baseline (speedup 1.0000x reference)
import jax
import jax.numpy as jnp
from jax.experimental import pallas as pl


def kernel(z_mantis, W1, b1, ln_g, ln_b, W2, b2, Wr, br, Wo, Wv, bv, Wm, bm, scale):
    raise NotImplementedError("write your pallas kernel here")


# speedup vs baseline: 2.5924x; 2.5924x over previous
import functools

import jax
import jax.numpy as jnp
from jax.experimental import pallas as pl
from jax.experimental.pallas import tpu as pltpu

_LN_EPS = 1e-5
_NS_ITERS = 5


def _fwd_kernel(z_ref, w1_ref, b1_ref, g_ref, bb_ref, w2_ref, wr_ref, bo_ref,
                x_ref, s1_ref, s2_ref):
    j = pl.program_id(1)
    z = z_ref[...]
    h = jnp.dot(z, w1_ref[...], preferred_element_type=jnp.float32) + b1_ref[...]
    m = jnp.mean(h, axis=-1, keepdims=True)
    hc = h - m
    v = jnp.mean(hc * hc, axis=-1, keepdims=True)
    h = hc * jax.lax.rsqrt(v + _LN_EPS) * g_ref[...] + bb_ref[...]
    h = 0.5 * h * (1.0 + jax.lax.erf(h * 0.7071067811865476))
    x = (jnp.dot(h, w2_ref[...], preferred_element_type=jnp.float32)
         + jnp.dot(z, wr_ref[...], preferred_element_type=jnp.float32)
         + bo_ref[...])
    x_ref[...] = x
    s1 = jnp.sum(x, axis=0, keepdims=True)
    s2 = jax.lax.dot_general(x, x, (((0,), (0,)), ((), ())),
                             preferred_element_type=jnp.float32)

    @pl.when(j == 0)
    def _init():
        s1_ref[...] = jnp.zeros_like(s1_ref)
        s2_ref[...] = jnp.zeros_like(s2_ref)

    s1_ref[...] += jnp.broadcast_to(s1[None], s1_ref.shape)
    s2_ref[...] += s2[None]


def _ns_kernel(s1_ref, s2_ref, wv_ref, bv_ref, wm_ref, bm_ref, sc_ref,
               m2_ref, c2_ref, *, nrows, cores, d_out):
    s1 = s1_ref[0]
    s2 = s2_ref[0]
    for k in range(1, cores):
        s1 = s1 + s1_ref[k]
        s2 = s2 + s2_ref[k]
    s1 = s1[0:1, :]
    nb = jnp.float32(nrows)
    mu = s1 / nb
    outer = jax.lax.dot_general(mu, mu, (((0,), (0,)), ((), ())),
                                preferred_element_type=jnp.float32)
    denom = jnp.float32(nrows - 1 if nrows > 1 else 1)
    ii = jax.lax.broadcasted_iota(jnp.int32, (d_out, d_out), 0)
    jj = jax.lax.broadcasted_iota(jnp.int32, (d_out, d_out), 1)
    eye = jnp.where(ii == jj, 1.0, 0.0).astype(jnp.float32)
    sigma = (s2 - nb * outer) / denom + 0.001 * eye
    tr = jnp.sum(sigma * eye)
    sn = tr * 1.5 + 1e-6
    ss = sigma / sn
    w = eye
    for _ in range(_NS_ITERS):
        t = jnp.dot(w, ss, preferred_element_type=jnp.float32)
        p = jax.lax.dot_general(t, w, (((1,), (1,)), ((), ())),
                                preferred_element_type=jnp.float32)
        w = jnp.dot(1.5 * eye - 0.5 * p, w, preferred_element_type=jnp.float32)
    a = w / jnp.sqrt(sn)
    mv = jax.lax.dot_general(a, wv_ref[...], (((0,), (1,)), ((), ())),
                             preferred_element_type=jnp.float32)
    mm = jax.lax.dot_general(a, wm_ref[...], (((0,), (1,)), ((), ())),
                             preferred_element_type=jnp.float32)
    cv = bv_ref[...] - jnp.dot(mu, mv, preferred_element_type=jnp.float32)
    cm = bm_ref[...] - jnp.dot(mu, mm, preferred_element_type=jnp.float32)
    sc = sc_ref[0, 0]
    m2_ref[...] = jnp.concatenate([mv * sc, mm], axis=1)
    c2 = jnp.concatenate([cv * sc, cm], axis=1)
    c2_ref[...] = jnp.broadcast_to(c2, c2_ref.shape)


def _apply_kernel(x_ref, m2_ref, c2_ref, o_ref, *, d_out):
    y = jnp.dot(x_ref[...], m2_ref[...],
                preferred_element_type=jnp.float32) + c2_ref[0:1, :]
    o_ref[...] = y[:, :d_out] * jax.nn.sigmoid(y[:, d_out:])


def kernel(z_mantis, W1, b1, ln_g, ln_b, W2, b2, Wr, br, Wo, Wv, bv, Wm, bm, scale):
    B, d_in = z_mantis.shape
    d_hid = W1.shape[0]
    d_out = W2.shape[0]
    f32 = jnp.float32

    w1t = W1.T
    w2t = W2.T @ Wo.T
    wrt = Wr.T @ Wo.T
    bo = ((b2 + br) @ Wo.T).reshape(1, d_out)
    b1r = b1.reshape(1, d_hid)
    gr = ln_g.reshape(1, d_hid)
    lbr = ln_b.reshape(1, d_hid)
    bvr = bv.reshape(1, d_out)
    bmr = bm.reshape(1, d_out)
    scr = scale.reshape(1, 1)

    cores = 2 if B % 2 == 0 else 1
    rows = B // cores
    bt = 2048 if rows % 2048 == 0 else rows
    nb = rows // bt

    x, s1, s2 = pl.pallas_call(
        _fwd_kernel,
        grid=(cores, nb),
        in_specs=[
            pl.BlockSpec((bt, d_in), lambda c, j: (c * nb + j, 0)),
            pl.BlockSpec((d_in, d_hid), lambda c, j: (0, 0)),
            pl.BlockSpec((1, d_hid), lambda c, j: (0, 0)),
            pl.BlockSpec((1, d_hid), lambda c, j: (0, 0)),
            pl.BlockSpec((1, d_hid), lambda c, j: (0, 0)),
            pl.BlockSpec((d_hid, d_out), lambda c, j: (0, 0)),
            pl.BlockSpec((d_in, d_out), lambda c, j: (0, 0)),
            pl.BlockSpec((1, d_out), lambda c, j: (0, 0)),
        ],
        out_specs=[
            pl.BlockSpec((bt, d_out), lambda c, j: (c * nb + j, 0)),
            pl.BlockSpec((1, 8, d_out), lambda c, j: (c, 0, 0)),
            pl.BlockSpec((1, d_out, d_out), lambda c, j: (c, 0, 0)),
        ],
        out_shape=[
            jax.ShapeDtypeStruct((B, d_out), f32),
            jax.ShapeDtypeStruct((cores, 8, d_out), f32),
            jax.ShapeDtypeStruct((cores, d_out, d_out), f32),
        ],
        compiler_params=pltpu.CompilerParams(
            dimension_semantics=("parallel", "arbitrary")),
        name="rcwa_fwd_stats",
    )(z_mantis, w1t, b1r, gr, lbr, w2t, wrt, bo)

    m2, c2 = pl.pallas_call(
        functools.partial(_ns_kernel, nrows=B, cores=cores, d_out=d_out),
        out_shape=[
            jax.ShapeDtypeStruct((d_out, 2 * d_out), f32),
            jax.ShapeDtypeStruct((8, 2 * d_out), f32),
        ],
        name="rcwa_ns_solve",
    )(s1, s2, Wv, bvr, Wm, bmr, scr)

    bt3 = 8192 if rows % 8192 == 0 else bt
    nb3 = rows // bt3
    out = pl.pallas_call(
        functools.partial(_apply_kernel, d_out=d_out),
        grid=(cores, nb3),
        in_specs=[
            pl.BlockSpec((bt3, d_out), lambda c, j: (c * nb3 + j, 0)),
            pl.BlockSpec((d_out, 2 * d_out), lambda c, j: (0, 0)),
            pl.BlockSpec((8, 2 * d_out), lambda c, j: (0, 0)),
        ],
        out_specs=pl.BlockSpec((bt3, d_out), lambda c, j: (c * nb3 + j, 0)),
        out_shape=jax.ShapeDtypeStruct((B, d_out), f32),
        compiler_params=pltpu.CompilerParams(
            dimension_semantics=("parallel", "arbitrary")),
        name="rcwa_apply",
    )(x, m2, c2)
    return out


# speedup vs baseline: 2.5958x; 1.0013x over previous
import functools

import jax
import jax.numpy as jnp
from jax.experimental import pallas as pl
from jax.experimental.pallas import tpu as pltpu

_LN_EPS = 1e-5
_NS_ITERS = 5


def _fwd_kernel(z_ref, w1_ref, b1_ref, g_ref, bb_ref, w2_ref, wr_ref, bo_ref,
                x_ref, s1_ref, s2_ref):
    j = pl.program_id(0)
    z = z_ref[...]
    h = jnp.dot(z, w1_ref[...], preferred_element_type=jnp.float32) + b1_ref[...]
    m = jnp.mean(h, axis=-1, keepdims=True)
    hc = h - m
    v = jnp.mean(hc * hc, axis=-1, keepdims=True)
    h = hc * jax.lax.rsqrt(v + _LN_EPS) * g_ref[...] + bb_ref[...]
    h = 0.5 * h * (1.0 + jax.lax.erf(h * 0.7071067811865476))
    x = (jnp.dot(h, w2_ref[...], preferred_element_type=jnp.float32)
         + jnp.dot(z, wr_ref[...], preferred_element_type=jnp.float32)
         + bo_ref[...])
    x_ref[...] = x
    s1 = jnp.sum(x, axis=0, keepdims=True)
    s2 = jax.lax.dot_general(x, x, (((0,), (0,)), ((), ())),
                             preferred_element_type=jnp.float32)

    @pl.when(j == 0)
    def _init():
        s1_ref[...] = jnp.zeros_like(s1_ref)
        s2_ref[...] = jnp.zeros_like(s2_ref)

    s1_ref[...] += jnp.broadcast_to(s1, s1_ref.shape)
    s2_ref[...] += s2


def _ns_kernel(s1_ref, s2_ref, wv_ref, bv_ref, wm_ref, bm_ref, sc_ref,
               m2_ref, c2_ref, *, nrows, d_out):
    s1 = s1_ref[0:1, :]
    s2 = s2_ref[...]
    nb = jnp.float32(nrows)
    mu = s1 / nb
    outer = jax.lax.dot_general(mu, mu, (((0,), (0,)), ((), ())),
                                preferred_element_type=jnp.float32)
    denom = jnp.float32(nrows - 1 if nrows > 1 else 1)
    ii = jax.lax.broadcasted_iota(jnp.int32, (d_out, d_out), 0)
    jj = jax.lax.broadcasted_iota(jnp.int32, (d_out, d_out), 1)
    eye = jnp.where(ii == jj, 1.0, 0.0).astype(jnp.float32)
    sigma = (s2 - nb * outer) / denom + 0.001 * eye
    tr = jnp.sum(sigma * eye)
    sn = tr * 1.5 + 1e-6
    ss = sigma / sn
    w = eye
    for _ in range(_NS_ITERS):
        t = jnp.dot(w, ss, preferred_element_type=jnp.float32)
        p = jax.lax.dot_general(t, w, (((1,), (1,)), ((), ())),
                                preferred_element_type=jnp.float32)
        w = jnp.dot(1.5 * eye - 0.5 * p, w, preferred_element_type=jnp.float32)
    a = w / jnp.sqrt(sn)
    mv = jax.lax.dot_general(a, wv_ref[...], (((0,), (1,)), ((), ())),
                             preferred_element_type=jnp.float32)
    mm = jax.lax.dot_general(a, wm_ref[...], (((0,), (1,)), ((), ())),
                             preferred_element_type=jnp.float32)
    cv = bv_ref[...] - jnp.dot(mu, mv, preferred_element_type=jnp.float32)
    cm = bm_ref[...] - jnp.dot(mu, mm, preferred_element_type=jnp.float32)
    sc = sc_ref[0, 0]
    m2_ref[...] = jnp.concatenate([mv * sc, mm], axis=1)
    c2 = jnp.concatenate([cv * sc, cm], axis=1)
    c2_ref[...] = jnp.broadcast_to(c2, c2_ref.shape)


def _apply_kernel(x_ref, m2_ref, c2_ref, o_ref, *, d_out):
    y = jnp.dot(x_ref[...], m2_ref[...],
                preferred_element_type=jnp.float32) + c2_ref[0:1, :]
    o_ref[...] = y[:, :d_out] * jax.nn.sigmoid(y[:, d_out:])


def kernel(z_mantis, W1, b1, ln_g, ln_b, W2, b2, Wr, br, Wo, Wv, bv, Wm, bm, scale):
    B, d_in = z_mantis.shape
    d_hid = W1.shape[0]
    d_out = W2.shape[0]
    f32 = jnp.float32

    w1t = W1.T
    w2t = W2.T @ Wo.T
    wrt = Wr.T @ Wo.T
    bo = ((b2 + br) @ Wo.T).reshape(1, d_out)
    b1r = b1.reshape(1, d_hid)
    gr = ln_g.reshape(1, d_hid)
    lbr = ln_b.reshape(1, d_hid)
    bvr = bv.reshape(1, d_out)
    bmr = bm.reshape(1, d_out)
    scr = scale.reshape(1, 1)

    bt = 2048 if B % 2048 == 0 else B
    nb = B // bt

    x, s1, s2 = pl.pallas_call(
        _fwd_kernel,
        grid=(nb,),
        in_specs=[
            pl.BlockSpec((bt, d_in), lambda j: (j, 0)),
            pl.BlockSpec((d_in, d_hid), lambda j: (0, 0)),
            pl.BlockSpec((1, d_hid), lambda j: (0, 0)),
            pl.BlockSpec((1, d_hid), lambda j: (0, 0)),
            pl.BlockSpec((1, d_hid), lambda j: (0, 0)),
            pl.BlockSpec((d_hid, d_out), lambda j: (0, 0)),
            pl.BlockSpec((d_in, d_out), lambda j: (0, 0)),
            pl.BlockSpec((1, d_out), lambda j: (0, 0)),
        ],
        out_specs=[
            pl.BlockSpec((bt, d_out), lambda j: (j, 0)),
            pl.BlockSpec((8, d_out), lambda j: (0, 0)),
            pl.BlockSpec((d_out, d_out), lambda j: (0, 0)),
        ],
        out_shape=[
            jax.ShapeDtypeStruct((B, d_out), f32),
            jax.ShapeDtypeStruct((8, d_out), f32),
            jax.ShapeDtypeStruct((d_out, d_out), f32),
        ],
        compiler_params=pltpu.CompilerParams(
            dimension_semantics=("arbitrary",)),
        name="rcwa_fwd_stats",
    )(z_mantis, w1t, b1r, gr, lbr, w2t, wrt, bo)

    m2, c2 = pl.pallas_call(
        functools.partial(_ns_kernel, nrows=B, d_out=d_out),
        out_shape=[
            jax.ShapeDtypeStruct((d_out, 2 * d_out), f32),
            jax.ShapeDtypeStruct((8, 2 * d_out), f32),
        ],
        name="rcwa_ns_solve",
    )(s1, s2, Wv, bvr, Wm, bmr, scr)

    bt3 = 8192 if B % 8192 == 0 else bt
    nb3 = B // bt3
    out = pl.pallas_call(
        functools.partial(_apply_kernel, d_out=d_out),
        grid=(nb3,),
        in_specs=[
            pl.BlockSpec((bt3, d_out), lambda j: (j, 0)),
            pl.BlockSpec((d_out, 2 * d_out), lambda j: (0, 0)),
            pl.BlockSpec((8, 2 * d_out), lambda j: (0, 0)),
        ],
        out_specs=pl.BlockSpec((bt3, d_out), lambda j: (j, 0)),
        out_shape=jax.ShapeDtypeStruct((B, d_out), f32),
        compiler_params=pltpu.CompilerParams(
            dimension_semantics=("arbitrary",)),
        name="rcwa_apply",
    )(x, m2, c2)
    return out
